# baseline (speedup 1.0000x reference)
import functools

import jax
import jax.numpy as jnp
from jax import lax
from jax.experimental import pallas as pl
from jax.experimental.pallas import tpu as pltpu
from jax.experimental.pallas import tpu_sc as plsc

B = 4096
L = 200
LP = 208
HALF = LP // 2
D = 64
C = 2
NC = 2
NS = 16
NW = NC * NS
BPW = B // NW
LANES = 16


def _sc_embed_sums(ids_p, table):
    mesh = plsc.VectorSubcoreMesh(
        core_axis_name="c", subcore_axis_name="s",
        num_cores=NC, num_subcores=NS)

    @functools.partial(
        pl.kernel,
        out_type=jax.ShapeDtypeStruct((B, D), jnp.float32),
        mesh=mesh,
        scratch_types=[
            pltpu.VMEM((BPW, LP), jnp.int32),
            pltpu.VMEM((LP, D), jnp.float32),
            pltpu.VMEM((LP, D), jnp.float32),
            pltpu.VMEM((BPW, D), jnp.float32),
            pltpu.SemaphoreType.DMA,
            pltpu.SemaphoreType.DMA,
            pltpu.SemaphoreType.DMA,
            pltpu.SemaphoreType.DMA,
        ],
        compiler_params=pltpu.CompilerParams(use_tc_tiling_on_sc=False),
    )
    def k(ids_hbm, table_hbm, out_hbm, ids_v, buf_a, buf_b, sums_v,
          sa0, sa1, sb0, sb1):
        wid = lax.axis_index("s") * NC + lax.axis_index("c")
        base = wid * BPW
        pltpu.sync_copy(ids_hbm.at[pl.ds(base, BPW), :], ids_v)

        def copies(r, sbuf, s0, s1):
            return (
                pltpu.make_async_copy(
                    table_hbm.at[ids_v.at[r, pl.ds(0, HALF)]],
                    sbuf.at[pl.ds(0, HALF)], s0),
                pltpu.make_async_copy(
                    table_hbm.at[ids_v.at[r, pl.ds(HALF, HALF)]],
                    sbuf.at[pl.ds(HALF, HALF)], s1),
            )

        def issue(r, sbuf, s0, s1):
            c0, c1 = copies(r, sbuf, s0, s1)
            c0.start()
            c1.start()

        def wait(r, sbuf, s0, s1):
            c0, c1 = copies(r, sbuf, s0, s1)
            c0.wait()
            c1.wait()

        def accum(r, sbuf):
            z = jnp.zeros((LANES,), jnp.float32)

            def body(t, a):
                return tuple(
                    a[j] + sbuf[t, pl.ds(LANES * j, LANES)]
                    for j in range(D // LANES))

            a = lax.fori_loop(0, LP, body, (z,) * (D // LANES), unroll=8)
            for j in range(D // LANES):
                sums_v[r, pl.ds(LANES * j, LANES)] = a[j]

        issue(0, buf_a, sa0, sa1)

        def step(i, carry):
            rr = 2 * i
            issue(rr + 1, buf_b, sb0, sb1)
            wait(rr, buf_a, sa0, sa1)
            accum(rr, buf_a)

            @pl.when(rr + 2 < BPW)
            def _():
                issue(rr + 2, buf_a, sa0, sa1)

            wait(rr + 1, buf_b, sb0, sb1)
            accum(rr + 1, buf_b)
            return carry

        lax.fori_loop(0, BPW // 2, step, 0)
        pltpu.sync_copy(sums_v, out_hbm.at[pl.ds(base, BPW), :])

    return k(ids_p, table)


def _tc_head(ids, sums, w8, b8):
    R = 512

    def body(ids_ref, sums_ref, w_ref, b_ref, out_ref):
        cnt = jnp.sum((ids_ref[...] != 0).astype(jnp.float32),
                      axis=1, keepdims=True)
        sent = sums_ref[...] / (cnt + 1e-8)
        out_ref[...] = lax.dot_general(
            sent, w_ref[...], (((1,), (1,)), ((), ())),
            preferred_element_type=jnp.float32) + b_ref[...]

    return pl.pallas_call(
        body,
        grid=(B // R,),
        in_specs=[
            pl.BlockSpec((R, L), lambda i: (i, 0)),
            pl.BlockSpec((R, D), lambda i: (i, 0)),
            pl.BlockSpec((8, D), lambda i: (0, 0)),
            pl.BlockSpec((1, 8), lambda i: (0, 0)),
        ],
        out_specs=pl.BlockSpec((R, 8), lambda i: (i, 0)),
        out_shape=jax.ShapeDtypeStruct((B, 8), jnp.float32),
    )(ids, sums, w8, b8)


def kernel(input_ids, table, W, b):
    ids = input_ids.astype(jnp.int32)
    ids_p = jnp.pad(ids, ((0, 0), (0, LP - L)))
    sums = _sc_embed_sums(ids_p, table.astype(jnp.float32))
    w8 = jnp.pad(W.astype(jnp.float32), ((0, 8 - C), (0, 0)))
    b8 = jnp.pad(b.astype(jnp.float32), (0, 8 - C)).reshape(1, 8)
    logits8 = _tc_head(ids, sums, w8, b8)
    return logits8[:, :C]


# speedup vs baseline: 1.8469x; 1.8469x over previous
import functools

import jax
import jax.numpy as jnp
from jax import lax
from jax.experimental import pallas as pl
from jax.experimental.pallas import tpu as pltpu
from jax.experimental.pallas import tpu_sc as plsc

B = 4096
L = 200
LP = 208
HALF = LP // 2
D = 64
C = 2
NC = 2
NS = 16
NW = NC * NS
BPW = B // NW
LANES = 16


def _sc_embed_sums(ids_p, table):
    mesh = plsc.VectorSubcoreMesh(
        core_axis_name="c", subcore_axis_name="s",
        num_cores=NC, num_subcores=NS)

    @functools.partial(
        pl.kernel,
        out_type=jax.ShapeDtypeStruct((B, D), jnp.float32),
        mesh=mesh,
        scratch_types=[
            pltpu.VMEM((BPW, LP), jnp.int32),
            pltpu.VMEM((LP, D), jnp.float32),
            pltpu.VMEM((LP, D), jnp.float32),
            pltpu.VMEM((BPW, D), jnp.float32),
            pltpu.SemaphoreType.DMA,
            pltpu.SemaphoreType.DMA,
            pltpu.SemaphoreType.DMA,
            pltpu.SemaphoreType.DMA,
        ],
        compiler_params=pltpu.CompilerParams(use_tc_tiling_on_sc=False),
    )
    def k(ids_hbm, table_hbm, out_hbm, ids_v, buf_a, buf_b, sums_v,
          sa0, sa1, sb0, sb1):
        wid = lax.axis_index("s") * NC + lax.axis_index("c")
        base = wid * BPW
        pltpu.sync_copy(ids_hbm.at[pl.ds(base, BPW), :], ids_v)

        def copies(r, sbuf, s0, s1):
            return (
                pltpu.make_async_copy(
                    table_hbm.at[ids_v.at[r, pl.ds(0, HALF)]],
                    sbuf.at[pl.ds(0, HALF)], s0),
                pltpu.make_async_copy(
                    table_hbm.at[ids_v.at[r, pl.ds(HALF, HALF)]],
                    sbuf.at[pl.ds(HALF, HALF)], s1),
            )

        def issue(r, sbuf, s0, s1):
            c0, c1 = copies(r, sbuf, s0, s1)
            c0.start()
            c1.start()

        def wait(r, sbuf, s0, s1):
            c0, c1 = copies(r, sbuf, s0, s1)
            c0.wait()
            c1.wait()

        def accum(r, sbuf):
            z = jnp.zeros((LANES,), jnp.float32)

            def body(t, a):
                return tuple(
                    a[j] + sbuf[t, pl.ds(LANES * j, LANES)]
                    for j in range(D // LANES))

            a = lax.fori_loop(0, L, body, (z,) * (D // LANES), unroll=8)
            for j in range(D // LANES):
                sums_v[r, pl.ds(LANES * j, LANES)] = a[j]

        issue(0, buf_a, sa0, sa1)

        def step(i, carry):
            rr = 2 * i
            issue(rr + 1, buf_b, sb0, sb1)
            wait(rr, buf_a, sa0, sa1)
            accum(rr, buf_a)

            @pl.when(rr + 2 < BPW)
            def _():
                issue(rr + 2, buf_a, sa0, sa1)

            wait(rr + 1, buf_b, sb0, sb1)
            accum(rr + 1, buf_b)
            return carry

        lax.fori_loop(0, BPW // 2, step, 0)
        pltpu.sync_copy(sums_v, out_hbm.at[pl.ds(base, BPW), :])

    return k(ids_p, table)


def _tc_head(ids, sums, w8, b8):
    R = 512

    def body(ids_ref, sums_ref, w_ref, b_ref, out_ref):
        cnt = jnp.sum((ids_ref[...] != 0).astype(jnp.float32),
                      axis=1, keepdims=True)
        sent = sums_ref[...] / (cnt + 1e-8)
        out_ref[...] = lax.dot_general(
            sent, w_ref[...], (((1,), (1,)), ((), ())),
            preferred_element_type=jnp.float32) + b_ref[...]

    return pl.pallas_call(
        body,
        grid=(B // R,),
        in_specs=[
            pl.BlockSpec((R, L), lambda i: (i, 0)),
            pl.BlockSpec((R, D), lambda i: (i, 0)),
            pl.BlockSpec((8, D), lambda i: (0, 0)),
            pl.BlockSpec((1, 8), lambda i: (0, 0)),
        ],
        out_specs=pl.BlockSpec((R, 8), lambda i: (i, 0)),
        out_shape=jax.ShapeDtypeStruct((B, 8), jnp.float32),
    )(ids, sums, w8, b8)


def kernel(input_ids, table, W, b):
    ids = input_ids.astype(jnp.int32)
    npad = LP - L
    pad_ids = (jnp.arange(B * npad, dtype=jnp.int32).reshape(B, npad)
               * 997) % table.shape[0]
    ids_p = jnp.concatenate([ids, pad_ids], axis=1)
    sums = _sc_embed_sums(ids_p, table.astype(jnp.float32))
    w8 = jnp.pad(W.astype(jnp.float32), ((0, 8 - C), (0, 0)))
    b8 = jnp.pad(b.astype(jnp.float32), (0, 8 - C)).reshape(1, 8)
    logits8 = _tc_head(ids, sums, w8, b8)
    return logits8[:, :C]
